# baseline (speedup 1.0000x reference)
import functools

import jax
import jax.numpy as jnp
from jax import lax
from jax.experimental import pallas as pl
from jax.experimental.pallas import tpu as pltpu
from jax.experimental.pallas import tpu_sc as plsc

EMB = 128
CHUNK = 200
IB = 16
NBUF = 4


def _sc_embedding_lookup(x_flat, table):
    n = x_flat.shape[0]
    n_rows = table.shape[0]
    info = plsc.get_sparse_core_info()
    nw = info.num_cores * info.num_subcores
    per_w = n // nw
    n_iters = per_w // CHUNK
    n_blocks = n_iters // IB
    assert per_w % CHUNK == 0 and n % nw == 0 and n_iters % IB == 0
    assert IB % NBUF == 0 and n_blocks % 2 == 0

    mesh = plsc.VectorSubcoreMesh(core_axis_name="c", subcore_axis_name="s")

    @functools.partial(
        pl.kernel,
        mesh=mesh,
        compiler_params=pltpu.CompilerParams(needs_layout_passes=False),
        out_type=jax.ShapeDtypeStruct((n, EMB), jnp.float32),
        scratch_types=[
            pltpu.VMEM_SHARED((n_rows, EMB), jnp.float32),
            pltpu.VMEM((IB * CHUNK,), jnp.int32),
            pltpu.VMEM((IB * CHUNK,), jnp.int32),
        ]
        + [pltpu.VMEM((CHUNK, EMB), jnp.float32) for _ in range(NBUF)]
        + [
            pltpu.SemaphoreType.DMA,
            pltpu.SemaphoreType.DMA,
            pltpu.SemaphoreType.DMA,
            pltpu.SemaphoreType.DMA,
        ],
    )
    def k(x_hbm, table_hbm, out_hbm, table_v, idx0, idx1, *rest):
        rbufs = rest[:NBUF]
        sem_i0, sem_i1, sem_g, sem_s = rest[NBUF:]
        wid = lax.axis_index("s") * info.num_cores + lax.axis_index("c")
        base = wid * per_w

        @pl.when(lax.axis_index("s") == 0)
        def _():
            pltpu.sync_copy(table_hbm, table_v)

        plsc.subcore_barrier()

        ibufs = (idx0, idx1)
        isems = (sem_i0, sem_i1)

        pltpu.async_copy(x_hbm.at[pl.ds(base, IB * CHUNK)], idx0, sem_i0)

        def do_block(p, parity, ib, isem):
            b = p * 2 + parity
            blk_start = base + b * IB * CHUNK

            @pl.when(b + 1 < n_blocks)
            def _():
                pltpu.async_copy(
                    x_hbm.at[pl.ds(blk_start + IB * CHUNK, IB * CHUNK)],
                    ibufs[1 - parity],
                    isems[1 - parity],
                )

            pltpu.make_async_copy(
                x_hbm.at[pl.ds(blk_start, IB * CHUNK)], ib, isem
            ).wait()

            for c in range(IB):
                buf = rbufs[c % NBUF]
                start = blk_start + c * CHUNK

                def drain_one():
                    pltpu.make_async_copy(
                        buf, out_hbm.at[pl.ds(base, CHUNK)], sem_s
                    ).wait()

                if c >= NBUF or parity == 1:
                    drain_one()
                else:
                    pl.when(p > 0)(drain_one)

                pltpu.async_copy(
                    table_v.at[ib.at[pl.ds(c * CHUNK, CHUNK)]], buf, sem_g
                ).wait()

                pltpu.async_copy(buf, out_hbm.at[pl.ds(start, CHUNK)], sem_s)

        def body(p, carry):
            do_block(p, 0, idx0, sem_i0)
            do_block(p, 1, idx1, sem_i1)
            return carry

        lax.fori_loop(0, n_blocks // 2, body, 0)
        for _ in range(NBUF):
            pltpu.make_async_copy(
                rbufs[0], out_hbm.at[pl.ds(base, CHUNK)], sem_s
            ).wait()

    return k(x_flat, table)


def kernel(x, table):
    b, h = x.shape
    out = _sc_embedding_lookup(x.reshape(b * h), table)
    return out.reshape(b, h, EMB)


# speedup vs baseline: 1.0066x; 1.0066x over previous
import functools

import jax
import jax.numpy as jnp
from jax import lax
from jax.experimental import pallas as pl
from jax.experimental.pallas import tpu as pltpu
from jax.experimental.pallas import tpu_sc as plsc

EMB = 128
CHUNK = 400
IB = 16


def _sc_embedding_lookup(x_flat, table):
    n = x_flat.shape[0]
    n_rows = table.shape[0]
    info = plsc.get_sparse_core_info()
    nw = info.num_cores * info.num_subcores
    per_w = n // nw
    n_iters = per_w // CHUNK
    n_blocks = n_iters // IB
    assert per_w % CHUNK == 0 and n % nw == 0 and n_iters % IB == 0
    assert IB % 2 == 0 and n_blocks % 2 == 0

    mesh = plsc.VectorSubcoreMesh(core_axis_name="c", subcore_axis_name="s")

    @functools.partial(
        pl.kernel,
        mesh=mesh,
        compiler_params=pltpu.CompilerParams(needs_layout_passes=False),
        out_type=jax.ShapeDtypeStruct((n, EMB), jnp.float32),
        scratch_types=[
            pltpu.VMEM_SHARED((n_rows, EMB), jnp.float32),
            pltpu.VMEM((IB * CHUNK,), jnp.int32),
            pltpu.VMEM((IB * CHUNK,), jnp.int32),
            pltpu.VMEM((CHUNK, EMB), jnp.float32),
            pltpu.VMEM((CHUNK, EMB), jnp.float32),
            pltpu.SemaphoreType.DMA,
            pltpu.SemaphoreType.DMA,
            pltpu.SemaphoreType.DMA,
            pltpu.SemaphoreType.DMA,
        ],
    )
    def k(x_hbm, table_hbm, out_hbm, table_v, idx0, idx1, rows0, rows1,
          sem_i0, sem_i1, sem_g, sem_s):
        wid = lax.axis_index("s") * info.num_cores + lax.axis_index("c")
        base = wid * per_w

        @pl.when(lax.axis_index("s") == 0)
        def _():
            pltpu.sync_copy(table_hbm, table_v)

        plsc.subcore_barrier()

        rbufs = (rows0, rows1)
        ibufs = (idx0, idx1)
        isems = (sem_i0, sem_i1)

        pltpu.async_copy(x_hbm.at[pl.ds(base, IB * CHUNK)], idx0, sem_i0)

        def do_block(p, parity, ib, isem):
            b = p * 2 + parity
            blk_start = base + b * IB * CHUNK

            @pl.when(b + 1 < n_blocks)
            def _():
                pltpu.async_copy(
                    x_hbm.at[pl.ds(blk_start + IB * CHUNK, IB * CHUNK)],
                    ibufs[1 - parity],
                    isems[1 - parity],
                )

            pltpu.make_async_copy(
                x_hbm.at[pl.ds(blk_start, IB * CHUNK)], ib, isem
            ).wait()

            for c in range(IB):
                i = b * IB + c
                buf = rbufs[c % 2]
                start = blk_start + c * CHUNK

                pltpu.async_copy(
                    table_v.at[ib.at[pl.ds(c * CHUNK, CHUNK)]], buf, sem_g
                ).wait()

                def drain_prev():
                    pltpu.make_async_copy(
                        rbufs[1 - (c % 2)],
                        out_hbm.at[pl.ds(base, CHUNK)],
                        sem_s,
                    ).wait()

                if c == 0 and parity == 0:
                    pl.when(p > 0)(drain_prev)
                else:
                    drain_prev()

                pltpu.async_copy(buf, out_hbm.at[pl.ds(start, CHUNK)], sem_s)

        def body(p, carry):
            do_block(p, 0, idx0, sem_i0)
            do_block(p, 1, idx1, sem_i1)
            return carry

        lax.fori_loop(0, n_blocks // 2, body, 0)
        pltpu.make_async_copy(
            rbufs[(n_iters - 1) % 2], out_hbm.at[pl.ds(base, CHUNK)], sem_s
        ).wait()

    return k(x_flat, table)


def kernel(x, table):
    b, h = x.shape
    out = _sc_embedding_lookup(x.reshape(b * h), table)
    return out.reshape(b, h, EMB)
